# baseline (speedup 1.0000x reference)
import functools

import jax
import jax.numpy as jnp
from jax import lax
from jax.experimental import pallas as pl
from jax.experimental.pallas import tpu as pltpu
from jax.experimental.pallas import tpu_sc as plsc

N_NODES = 10000
N_EDGES = 160000
D = 128
G = 8
NC, NS, L = 2, 16, 16
NW = NC * NS
NPW = 320
N_PAD = NW * NPW
CH = 2048
NCHUNK = N_EDGES // CH
REM = N_EDGES - NCHUNK * CH
GB = 128
POOLN = N_EDGES + 512
DUMP = N_EDGES + 384


def _mm_body(x_ref, w_ref, b_ref, o_ref):
    o_ref[...] = (
        jnp.dot(x_ref[...], w_ref[...], preferred_element_type=jnp.float32)
        + b_ref[...]
    )


def _linear(xt, W, b):
    rows = xt.shape[0]
    BN = 1000
    return pl.pallas_call(
        _mm_body,
        grid=(rows // BN,),
        in_specs=[
            pl.BlockSpec((BN, D), lambda i: (i, 0)),
            pl.BlockSpec((D, D), lambda i: (0, 0)),
            pl.BlockSpec((1, D), lambda i: (0, 0)),
        ],
        out_specs=pl.BlockSpec((BN, D), lambda i: (i, 0)),
        out_shape=jax.ShapeDtypeStruct((rows, D), jnp.float32),
    )(xt, W, b.reshape(1, D))


def _sc_body(xf_hbm, srcg_hbm, dstg_hbm, out_hbm, fsrc_sh, fdst_sh, cnt_sh,
             src_ck, dst_ck, pos_buf, idx_buf, d_buf0, d_buf1, rows,
             out_loc, tmp, tmpc, semA, semB, semS):
    cid = lax.axis_index("c")
    sid = lax.axis_index("s")
    wid = sid * NC + cid
    lo = wid * NPW
    lane = lax.iota(jnp.int32, L)
    zero = jnp.zeros((L,), jnp.int32)

    tmp[pl.ds(0, L)] = zero
    tmp[pl.ds(2 * L, L)] = zero

    def start_d(c, s, sem):
        eb = pl.multiple_of(c * CH, 8)
        pltpu.async_copy(dstg_hbm.at[pl.ds(eb, CH)], dst_ck.at[s], sem)

    def wait_d(s, sem):
        pltpu.make_async_copy(dstg_hbm.at[pl.ds(0, CH)], dst_ck.at[s],
                              sem).wait()

    def count_grps(s, ngrp, acc):
        def grp(j, a):
            d = dst_ck[s, pl.ds(j * L, L)]
            m = (d >= lo) & (d < lo + NPW)
            return a + jnp.where(m, 1, 0)

        return lax.fori_loop(0, ngrp, grp, acc)

    start_d(0, 0, semA)

    def cpair(i, acc):
        wait_d(0, semA)
        start_d(2 * i + 1, 1, semB)
        acc = count_grps(0, CH // L, acc)
        wait_d(1, semB)
        start_d(jnp.minimum(2 * i + 2, NCHUNK - 1), 0, semA)
        return count_grps(1, CH // L, acc)

    acc = lax.fori_loop(0, NCHUNK // 2, cpair, zero)
    wait_d(0, semA)
    pltpu.sync_copy(dstg_hbm.at[pl.ds(NCHUNK * CH, REM)],
                    dst_ck.at[0, pl.ds(0, REM)])
    acc = count_grps(0, REM // L, acc)
    x = acc
    for sh in (1, 2, 4, 8):
        tmp[pl.ds(L, L)] = x
        x = x + tmp[pl.ds(L - sh, L)]
    cnt = x[L - 1]

    tmpc[pl.ds(0, L)] = jnp.full((L,), cnt, jnp.int32)
    pltpu.sync_copy(tmpc.at[pl.ds(0, L)], cnt_sh.at[pl.ds(sid * L, L)])
    plsc.subcore_barrier()
    pltpu.sync_copy(cnt_sh, tmpc)
    offv = zero
    for k in range(NS):
        ck = tmpc[pl.ds(k * L, L)]
        ck = (ck + 7) & ~7
        offv = offv + jnp.where(k < sid, ck, 0)
    off = pl.multiple_of(offv[0], 8)

    def start_sd(c, s, sem):
        eb = pl.multiple_of(c * CH, 8)
        pltpu.async_copy(srcg_hbm.at[pl.ds(eb, CH)], src_ck.at[s], sem)
        pltpu.async_copy(dstg_hbm.at[pl.ds(eb, CH)], dst_ck.at[s], sem)

    def wait_sd(s, sem):
        pltpu.make_async_copy(srcg_hbm.at[pl.ds(0, CH)], src_ck.at[s],
                              sem).wait()
        pltpu.make_async_copy(dstg_hbm.at[pl.ds(0, CH)], dst_ck.at[s],
                              sem).wait()

    def bucket_grps(s, ngrp, nslc, run):
        def grp2(jj, run):
            j0 = jj * 2
            j1 = jj * 2 + 1
            d0 = dst_ck[s, pl.ds(j0 * L, L)]
            d1 = dst_ck[s, pl.ds(j1 * L, L)]
            m0 = (d0 >= lo) & (d0 < lo + NPW)
            m1 = (d1 >= lo) & (d1 < lo + NPW)
            x0 = jnp.where(m0, 1, 0)
            x1 = jnp.where(m1, 1, 0)
            for sh in (1, 2, 4, 8):
                tmp[pl.ds(L, L)] = x0
                tmp[pl.ds(3 * L, L)] = x1
                x0 = x0 + tmp[pl.ds(L - sh, L)]
                x1 = x1 + tmp[pl.ds(3 * L - sh, L)]
            t0 = x0[L - 1]
            pos0 = jnp.where(m0, off + run + x0 - 1, DUMP + sid)
            pos1 = jnp.where(m1, off + run + t0 + x1 - 1, DUMP + sid)
            pos_buf[s, j0 // 8, pl.ds((j0 % 8) * L, L)] = pos0
            pos_buf[s, j1 // 8, pl.ds((j1 % 8) * L, L)] = pos1
            return run + t0 + x1[L - 1]

        run = lax.fori_loop(0, ngrp // 2, grp2, run)
        descs = []
        for t in range(nslc):
            descs.append(pltpu.async_copy(
                src_ck.at[s, pl.ds(t * 128, 128)],
                fsrc_sh.at[pos_buf.at[s, t]], semS))
            descs.append(pltpu.async_copy(
                dst_ck.at[s, pl.ds(t * 128, 128)],
                fdst_sh.at[pos_buf.at[s, t]], semS))
        for dsc in descs:
            dsc.wait()
        return run

    start_sd(0, 0, semA)

    def bpair(i, run):
        wait_sd(0, semA)
        start_sd(2 * i + 1, 1, semB)
        run = bucket_grps(0, CH // L, CH // 128, run)
        wait_sd(1, semB)
        start_sd(jnp.minimum(2 * i + 2, NCHUNK - 1), 0, semA)
        return bucket_grps(1, CH // L, CH // 128, run)

    run = lax.fori_loop(0, NCHUNK // 2, bpair, 0)
    wait_sd(0, semA)
    pltpu.sync_copy(srcg_hbm.at[pl.ds(NCHUNK * CH, REM)],
                    src_ck.at[0, pl.ds(0, REM)])
    pltpu.sync_copy(dstg_hbm.at[pl.ds(NCHUNK * CH, REM)],
                    dst_ck.at[0, pl.ds(0, REM)])
    bucket_grps(0, REM // L, REM // 128, run)

    neg = jnp.full((L,), -jnp.inf, jnp.float32)
    nsl = jnp.maximum((cnt + GB - 1) // GB, 1)
    nsl2 = (nsl + 1) // 2

    def graph_body(g, _):
        def load_clamp(t, s, d_buf):
            fb = pl.multiple_of(off + t * GB, 8)
            pltpu.sync_copy(fsrc_sh.at[pl.ds(fb, GB)], idx_buf.at[s])
            pltpu.sync_copy(fdst_sh.at[pl.ds(fb, GB)],
                            d_buf.at[pl.ds(0, GB)])
            for u in range(GB // L):
                seg = pl.ds(u * L, L)
                v = idx_buf[s, seg]
                idx_buf[s, seg] = jnp.clip(v, 0, N_NODES - 1) + g * N_NODES

        def start_gather(s, sem):
            return pltpu.async_copy(xf_hbm.at[idx_buf.at[s]], rows.at[s], sem)

        def wait_gather(s, sem):
            pltpu.make_async_copy(xf_hbm.at[idx_buf.at[s]], rows.at[s],
                                  sem).wait()

        def apply(t, s, d_buf):
            n = jnp.minimum(cnt - t * GB, GB)

            def gbody(q, _q):
                def one(e, ld):
                    cur = [out_loc[ld, pl.ds(k * L, L)]
                           for k in range(D // L)]
                    msg = [rows[s, e, pl.ds(k * L, L)]
                           for k in range(D // L)]
                    for k in range(D // L):
                        out_loc[ld, pl.ds(k * L, L)] = jnp.maximum(
                            cur[k], msg[k])

                for h in range(2):
                    base = q * 2 * L + h * L
                    dvec = d_buf[pl.ds(base, L)]
                    ldv = jnp.where(base + lane < n, dvec - lo, NPW)
                    for i in range(L):
                        one(base + i, ldv[i])
                return _q

            lax.fori_loop(0, (n + 2 * L - 1) // (2 * L), gbody, 0)

        load_clamp(0, 0, d_buf0)
        start_gather(0, semA)

        def initb(i, c):
            for k in range(D // L):
                out_loc[i, pl.ds(k * L, L)] = neg
            return c

        lax.fori_loop(0, NPW + 8, initb, 0)

        def pipe_body(t2, _p):
            tA = jnp.minimum(2 * t2, nsl - 1)
            tB = jnp.minimum(2 * t2 + 1, nsl - 1)
            tA2 = jnp.minimum(2 * t2 + 2, nsl - 1)
            load_clamp(tB, 1, d_buf1)
            start_gather(1, semB)
            wait_gather(0, semA)
            apply(tA, 0, d_buf0)
            load_clamp(tA2, 0, d_buf0)
            start_gather(0, semA)
            wait_gather(1, semB)
            apply(tB, 1, d_buf1)
            return _p

        lax.fori_loop(0, nsl2, pipe_body, 0)
        wait_gather(0, semA)

        def fixb(i, c):
            for k in range(D // L):
                seg = pl.ds(k * L, L)
                v = out_loc[i, seg]
                out_loc[i, seg] = jnp.where(v == -jnp.inf, 0.0, v)
            return c

        lax.fori_loop(0, NPW, fixb, 0)
        pltpu.sync_copy(out_loc.at[pl.ds(0, NPW), :],
                        out_hbm.at[g, pl.ds(lo, NPW), :])
        return _

    lax.fori_loop(0, G, graph_body, 0)


@functools.partial(
    pl.kernel,
    out_type=jax.ShapeDtypeStruct((G, N_PAD, D), jnp.float32),
    mesh=plsc.VectorSubcoreMesh(core_axis_name="c", subcore_axis_name="s"),
    scratch_types=[
        pltpu.VMEM_SHARED((POOLN,), jnp.int32),
        pltpu.VMEM_SHARED((POOLN,), jnp.int32),
        pltpu.VMEM_SHARED((NS * L,), jnp.int32),
        pltpu.VMEM((2, CH), jnp.int32),
        pltpu.VMEM((2, CH), jnp.int32),
        pltpu.VMEM((2, 16, 128), jnp.int32),
        pltpu.VMEM((2, GB), jnp.int32),
        pltpu.VMEM((GB + L,), jnp.int32),
        pltpu.VMEM((GB + L,), jnp.int32),
        pltpu.VMEM((2, GB, D), jnp.float32),
        pltpu.VMEM((NPW + 8, D), jnp.float32),
        pltpu.VMEM((4 * L,), jnp.int32),
        pltpu.VMEM((NS * L,), jnp.int32),
        pltpu.SemaphoreType.DMA,
        pltpu.SemaphoreType.DMA,
        pltpu.SemaphoreType.DMA,
    ],
)
def _sc_aggregate(xf_hbm, srcg_hbm, dstg_hbm, out_hbm, *scratch):
    _sc_body(xf_hbm, srcg_hbm, dstg_hbm, out_hbm, *scratch)


@jax.jit
def kernel(x, edge_index, W, b):
    x = x.astype(jnp.float32)
    B, N, F, T = x.shape
    xt = jnp.transpose(x, (0, 3, 1, 2)).reshape(B * T * N, F)
    xf = _linear(xt, W, b)
    out = _sc_aggregate(xf, edge_index[0], edge_index[1])
    final = out[:, :N_NODES, :].reshape(B, T, N, F).transpose(0, 2, 3, 1)
    return final


# speedup vs baseline: 1.0655x; 1.0655x over previous
import functools

import jax
import jax.numpy as jnp
from jax import lax
from jax.experimental import pallas as pl
from jax.experimental.pallas import tpu as pltpu
from jax.experimental.pallas import tpu_sc as plsc

N_NODES = 10000
N_EDGES = 160000
D = 128
G = 8
NC, NS, L = 2, 16, 16
NW = NC * NS
NPW = 320
N_PAD = NW * NPW
CH = 2048
NCHUNK = N_EDGES // CH
REM = N_EDGES - NCHUNK * CH
GB = 128
POOLN = N_EDGES + 512
DUMP = N_EDGES + 384


def _mm_body(x_ref, w_ref, b_ref, o_ref):
    o_ref[...] = (
        jnp.dot(x_ref[...], w_ref[...], preferred_element_type=jnp.float32)
        + b_ref[...]
    )


def _linear(xt, W, b):
    rows = xt.shape[0]
    BN = 1000
    return pl.pallas_call(
        _mm_body,
        grid=(rows // BN,),
        in_specs=[
            pl.BlockSpec((BN, D), lambda i: (i, 0)),
            pl.BlockSpec((D, D), lambda i: (0, 0)),
            pl.BlockSpec((1, D), lambda i: (0, 0)),
        ],
        out_specs=pl.BlockSpec((BN, D), lambda i: (i, 0)),
        out_shape=jax.ShapeDtypeStruct((rows, D), jnp.float32),
    )(xt, W, b.reshape(1, D))


def _sc_body(xf_hbm, srcg_hbm, dstg_hbm, out_hbm, fsrc_sh, fdst_sh, cnt_sh,
             src_ck, dst_ck, pos_buf, idx_buf, d_buf0, d_buf1, rows,
             out_loc, tmp, tmpc, semA, semB, semS):
    cid = lax.axis_index("c")
    sid = lax.axis_index("s")
    wid = sid * NC + cid
    lo = wid * NPW
    lane = lax.iota(jnp.int32, L)
    zero = jnp.zeros((L,), jnp.int32)

    tmp[pl.ds(0, L)] = zero
    tmp[pl.ds(2 * L, L)] = zero

    def start_d(c, s, sem):
        eb = pl.multiple_of(c * CH, 8)
        pltpu.async_copy(dstg_hbm.at[pl.ds(eb, CH)], dst_ck.at[s], sem)

    def wait_d(s, sem):
        pltpu.make_async_copy(dstg_hbm.at[pl.ds(0, CH)], dst_ck.at[s],
                              sem).wait()

    def count_grps(s, ngrp, acc):
        def grp(j, a):
            d = dst_ck[s, pl.ds(j * L, L)]
            m = (d >= lo) & (d < lo + NPW)
            return a + jnp.where(m, 1, 0)

        return lax.fori_loop(0, ngrp, grp, acc)

    start_d(0, 0, semA)

    def cpair(i, acc):
        wait_d(0, semA)
        start_d(2 * i + 1, 1, semB)
        acc = count_grps(0, CH // L, acc)
        wait_d(1, semB)
        start_d(jnp.minimum(2 * i + 2, NCHUNK - 1), 0, semA)
        return count_grps(1, CH // L, acc)

    acc = lax.fori_loop(0, NCHUNK // 2, cpair, zero)
    wait_d(0, semA)
    pltpu.sync_copy(dstg_hbm.at[pl.ds(NCHUNK * CH, REM)],
                    dst_ck.at[0, pl.ds(0, REM)])
    acc = count_grps(0, REM // L, acc)
    x = acc
    for sh in (1, 2, 4, 8):
        tmp[pl.ds(L, L)] = x
        x = x + tmp[pl.ds(L - sh, L)]
    cnt = x[L - 1]

    tmpc[pl.ds(0, L)] = jnp.full((L,), cnt, jnp.int32)
    pltpu.sync_copy(tmpc.at[pl.ds(0, L)], cnt_sh.at[pl.ds(sid * L, L)])
    plsc.subcore_barrier()
    pltpu.sync_copy(cnt_sh, tmpc)
    offv = zero
    for k in range(NS):
        ck = tmpc[pl.ds(k * L, L)]
        ck = (ck + 7) & ~7
        offv = offv + jnp.where(k < sid, ck, 0)
    off = pl.multiple_of(offv[0], 8)

    def start_sd(c, s, sem):
        eb = pl.multiple_of(c * CH, 8)
        pltpu.async_copy(srcg_hbm.at[pl.ds(eb, CH)], src_ck.at[s], sem)
        pltpu.async_copy(dstg_hbm.at[pl.ds(eb, CH)], dst_ck.at[s], sem)

    def wait_sd(s, sem):
        pltpu.make_async_copy(srcg_hbm.at[pl.ds(0, CH)], src_ck.at[s],
                              sem).wait()
        pltpu.make_async_copy(dstg_hbm.at[pl.ds(0, CH)], dst_ck.at[s],
                              sem).wait()

    def bucket_grps(s, ngrp, nslc, run):
        def grp2(jj, run):
            j0 = jj * 2
            j1 = jj * 2 + 1
            d0 = dst_ck[s, pl.ds(j0 * L, L)]
            d1 = dst_ck[s, pl.ds(j1 * L, L)]
            m0 = (d0 >= lo) & (d0 < lo + NPW)
            m1 = (d1 >= lo) & (d1 < lo + NPW)
            x0 = jnp.where(m0, 1, 0)
            x1 = jnp.where(m1, 1, 0)
            for sh in (1, 2, 4, 8):
                tmp[pl.ds(L, L)] = x0
                tmp[pl.ds(3 * L, L)] = x1
                x0 = x0 + tmp[pl.ds(L - sh, L)]
                x1 = x1 + tmp[pl.ds(3 * L - sh, L)]
            t0 = x0[L - 1]
            pos0 = jnp.where(m0, off + run + x0 - 1, DUMP + sid)
            pos1 = jnp.where(m1, off + run + t0 + x1 - 1, DUMP + sid)
            pos_buf[s, j0 // 8, pl.ds((j0 % 8) * L, L)] = pos0
            pos_buf[s, j1 // 8, pl.ds((j1 % 8) * L, L)] = pos1
            return run + t0 + x1[L - 1]

        run = lax.fori_loop(0, ngrp // 2, grp2, run)
        descs = []
        for t in range(nslc):
            descs.append(pltpu.async_copy(
                src_ck.at[s, pl.ds(t * 128, 128)],
                fsrc_sh.at[pos_buf.at[s, t]], semS))
            descs.append(pltpu.async_copy(
                dst_ck.at[s, pl.ds(t * 128, 128)],
                fdst_sh.at[pos_buf.at[s, t]], semS))
        for dsc in descs:
            dsc.wait()
        return run

    start_sd(0, 0, semA)

    def bpair(i, run):
        wait_sd(0, semA)
        start_sd(2 * i + 1, 1, semB)
        run = bucket_grps(0, CH // L, CH // 128, run)
        wait_sd(1, semB)
        start_sd(jnp.minimum(2 * i + 2, NCHUNK - 1), 0, semA)
        return bucket_grps(1, CH // L, CH // 128, run)

    run = lax.fori_loop(0, NCHUNK // 2, bpair, 0)
    wait_sd(0, semA)
    pltpu.sync_copy(srcg_hbm.at[pl.ds(NCHUNK * CH, REM)],
                    src_ck.at[0, pl.ds(0, REM)])
    pltpu.sync_copy(dstg_hbm.at[pl.ds(NCHUNK * CH, REM)],
                    dst_ck.at[0, pl.ds(0, REM)])
    bucket_grps(0, REM // L, REM // 128, run)

    neg = jnp.full((L,), -jnp.inf, jnp.float32)
    nsl = jnp.maximum((cnt + GB - 1) // GB, 1)
    nsl2 = (nsl + 1) // 2

    def graph_body(g, _):
        def load_clamp(t, s, d_buf):
            fb = pl.multiple_of(off + t * GB, 8)
            pltpu.sync_copy(fsrc_sh.at[pl.ds(fb, GB)], idx_buf.at[s])
            pltpu.sync_copy(fdst_sh.at[pl.ds(fb, GB)],
                            d_buf.at[pl.ds(0, GB)])
            for u in range(GB // L):
                seg = pl.ds(u * L, L)
                v = idx_buf[s, seg]
                idx_buf[s, seg] = jnp.clip(v, 0, N_NODES - 1) + g * N_NODES

        def start_gather(s, sem):
            return pltpu.async_copy(xf_hbm.at[idx_buf.at[s]], rows.at[s], sem)

        def wait_gather(s, sem):
            pltpu.make_async_copy(xf_hbm.at[idx_buf.at[s]], rows.at[s],
                                  sem).wait()

        def apply(t, s, d_buf):
            n = jnp.minimum(cnt - t * GB, GB)

            def gbody(q, _q):
                def one(e, ld):
                    cur = [out_loc[ld, pl.ds(k * L, L)]
                           for k in range(D // L)]
                    msg = [rows[s, e, pl.ds(k * L, L)]
                           for k in range(D // L)]
                    for k in range(D // L):
                        out_loc[ld, pl.ds(k * L, L)] = jnp.maximum(
                            cur[k], msg[k])

                base = q * L
                dvec = d_buf[pl.ds(base, L)]
                ldv = jnp.where(base + lane < n, dvec - lo, NPW)
                for i in range(L):
                    one(base + i, ldv[i])
                return _q

            lax.fori_loop(0, (n + L - 1) // L, gbody, 0)

        load_clamp(0, 0, d_buf0)
        start_gather(0, semA)

        def initb(i, c):
            for k in range(D // L):
                out_loc[i, pl.ds(k * L, L)] = neg
            return c

        lax.fori_loop(0, NPW + 8, initb, 0)

        def pipe_body(t2, _p):
            tA = jnp.minimum(2 * t2, nsl - 1)
            tB = jnp.minimum(2 * t2 + 1, nsl - 1)
            tA2 = jnp.minimum(2 * t2 + 2, nsl - 1)
            load_clamp(tB, 1, d_buf1)
            start_gather(1, semB)
            wait_gather(0, semA)
            apply(tA, 0, d_buf0)
            load_clamp(tA2, 0, d_buf0)
            start_gather(0, semA)
            wait_gather(1, semB)
            apply(tB, 1, d_buf1)
            return _p

        lax.fori_loop(0, nsl2, pipe_body, 0)
        wait_gather(0, semA)

        def fixb(i, c):
            for k in range(D // L):
                seg = pl.ds(k * L, L)
                v = out_loc[i, seg]
                out_loc[i, seg] = jnp.where(v == -jnp.inf, 0.0, v)
            return c

        lax.fori_loop(0, NPW, fixb, 0)
        pltpu.sync_copy(out_loc.at[pl.ds(0, NPW), :],
                        out_hbm.at[g, pl.ds(lo, NPW), :])
        return _

    lax.fori_loop(0, G, graph_body, 0)


@functools.partial(
    pl.kernel,
    out_type=jax.ShapeDtypeStruct((G, N_PAD, D), jnp.float32),
    mesh=plsc.VectorSubcoreMesh(core_axis_name="c", subcore_axis_name="s"),
    scratch_types=[
        pltpu.VMEM_SHARED((POOLN,), jnp.int32),
        pltpu.VMEM_SHARED((POOLN,), jnp.int32),
        pltpu.VMEM_SHARED((NS * L,), jnp.int32),
        pltpu.VMEM((2, CH), jnp.int32),
        pltpu.VMEM((2, CH), jnp.int32),
        pltpu.VMEM((2, 16, 128), jnp.int32),
        pltpu.VMEM((2, GB), jnp.int32),
        pltpu.VMEM((GB + L,), jnp.int32),
        pltpu.VMEM((GB + L,), jnp.int32),
        pltpu.VMEM((2, GB, D), jnp.float32),
        pltpu.VMEM((NPW + 8, D), jnp.float32),
        pltpu.VMEM((4 * L,), jnp.int32),
        pltpu.VMEM((NS * L,), jnp.int32),
        pltpu.SemaphoreType.DMA,
        pltpu.SemaphoreType.DMA,
        pltpu.SemaphoreType.DMA,
    ],
)
def _sc_aggregate(xf_hbm, srcg_hbm, dstg_hbm, out_hbm, *scratch):
    _sc_body(xf_hbm, srcg_hbm, dstg_hbm, out_hbm, *scratch)


@jax.jit
def kernel(x, edge_index, W, b):
    x = x.astype(jnp.float32)
    B, N, F, T = x.shape
    xt = jnp.transpose(x, (0, 3, 1, 2)).reshape(B * T * N, F)
    xf = _linear(xt, W, b)
    out = _sc_aggregate(xf, edge_index[0], edge_index[1])
    final = out[:, :N_NODES, :].reshape(B, T, N, F).transpose(0, 2, 3, 1)
    return final
